# Initial kernel scaffold; baseline (speedup 1.0000x reference)
import jax
import jax.numpy as jnp
from jax.experimental import pallas as pl


def kernel(graph_x, graph_edge_attr, buffer_edge_index, buffer_edge_attr, buffer_batch, comp, idx, mask, W, b, ln_gamma, ln_beta):
    raise NotImplementedError("write your pallas kernel here")


# speedup vs baseline: 2.7630x; 2.7630x over previous
import functools

import jax
import jax.numpy as jnp
from jax import lax
from jax.experimental import pallas as pl
from jax.experimental.pallas import tpu as pltpu
from jax.experimental.pallas import tpu_sc as plsc

NCO, NFI, EFI, ECO, D = 5000, 10000, 320000, 160000, 128
NSEG = 10240
EPAD = 327680
NW = 32
EPW = EPAD // NW
EROWS = EPAD // 128
RPW = EPW // 128
NSLAB = RPW // 8
NRT = NSEG // 16


def _mm_body(a_ref, w_ref, o_ref):
    o_ref[...] = jnp.dot(a_ref[...], w_ref[...],
                         preferred_element_type=jnp.float32,
                         precision=lax.Precision.HIGHEST)


def _matmul(a, w):
    m = a.shape[0]
    blk = 1000
    return pl.pallas_call(
        _mm_body,
        grid=(m // blk,),
        in_specs=[pl.BlockSpec((blk, D), lambda i: (i, 0)),
                  pl.BlockSpec((D, D), lambda i: (0, 0))],
        out_specs=pl.BlockSpec((blk, D), lambda i: (i, 0)),
        out_shape=jax.ShapeDtypeStruct((m, D), jnp.float32),
    )(a, w)


def _combine_body(sc_ref, xc_ref, b_ref, o_ref):
    cnt = jnp.maximum(sc_ref[1, :, 0:1], 1.0)
    o_ref[...] = sc_ref[0] / cnt + xc_ref[...] + b_ref[...]


def _combine(segcnt, xc, b2):
    blk = 1024
    return pl.pallas_call(
        _combine_body,
        grid=(NSEG // blk,),
        in_specs=[pl.BlockSpec((2, blk, D), lambda i: (0, i, 0)),
                  pl.BlockSpec((blk, D), lambda i: (i, 0)),
                  pl.BlockSpec((1, D), lambda i: (0, 0))],
        out_specs=pl.BlockSpec((blk, D), lambda i: (i, 0)),
        out_shape=jax.ShapeDtypeStruct((NSEG, D), jnp.float32),
    )(segcnt, xc, b2)


_SC_MESH = plsc.VectorSubcoreMesh(core_axis_name="c", subcore_axis_name="s")


def _gather_kernel_body(gx_hbm, xw_hbm, comp_hbm, xout_hbm, xc_hbm,
                        idxv, rows, sem):
    cid = lax.axis_index("c")
    sid = lax.axis_index("s")
    wid = sid * 2 + cid
    t = wid // 16
    u = wid % 16

    def gather_to(tab, out):
        pltpu.sync_copy(comp_hbm.at[pl.ds(8 * u, 8)], idxv)

        def body(j, _):
            pltpu.async_copy(tab.at[idxv.at[j]], rows, sem).wait()
            pltpu.sync_copy(rows, out.at[pl.ds(u * 1024 + j * 128, 128)])
            return 0
        lax.fori_loop(0, 8, body, 0)

    @pl.when(jnp.logical_and(t == 0, u < 10))
    def _():
        gather_to(gx_hbm, xout_hbm)

    @pl.when(jnp.logical_and(t == 1, u < 10))
    def _():
        gather_to(xw_hbm, xc_hbm)


def _scatter_kernel_body(ge_hbm, idx_hbm, dst_hbm, zseg_hbm, out_hbm,
                         tab_sh, idxv, dstv, rows, sem):
    cid = lax.axis_index("c")
    sid = lax.axis_index("s")
    r0 = sid * NRT
    pltpu.sync_copy(zseg_hbm.at[pl.ds(r0, NRT)], tab_sh.at[pl.ds(r0, NRT)])

    @pl.when(cid == 1)
    def _():
        one = jnp.ones((16,), jnp.float32)

        def orow(i, _):
            rows[lax.shift_right_logical(i, 3),
                 pl.ds(lax.mul(lax.rem(i, 8), 16), 16)] = one
            return 0
        lax.fori_loop(0, 1024, orow, 0)

    plsc.subcore_barrier()

    base = sid * (EROWS // 16)

    @pl.when(cid == 0)
    def _():
        def slab(k, _):
            pltpu.sync_copy(idx_hbm.at[pl.ds(base + 8 * k, 8)], idxv)
            pltpu.sync_copy(dst_hbm.at[pl.ds(base + 8 * k, 8)], dstv)

            def sub(j, _):
                pltpu.async_copy(ge_hbm.at[idxv.at[j]], rows, sem).wait()
                pltpu.sync_copy(rows, tab_sh.at[dstv.at[j]], add=True)
                return 0

            lax.fori_loop(0, 8, sub, 0)
            return 0

        lax.fori_loop(0, EROWS // 128, slab, 0)

    @pl.when(cid == 1)
    def _():
        def slab(k, _):
            pltpu.sync_copy(dst_hbm.at[pl.ds(base + 8 * k, 8)], dstv)

            def sub(j, _):
                pltpu.sync_copy(rows, tab_sh.at[dstv.at[j]], add=True)
                return 0

            lax.fori_loop(0, 8, sub, 0)
            return 0

        lax.fori_loop(0, EROWS // 128, slab, 0)

    plsc.subcore_barrier()
    pltpu.sync_copy(tab_sh.at[pl.ds(r0, NRT)], out_hbm.at[cid, pl.ds(r0, NRT)])


def _newton_rsqrt(x):
    i = lax.bitcast_convert_type(x, jnp.int32)
    i = jnp.int32(0x5F3759DF) - lax.shift_right_logical(i, 1)
    y = lax.bitcast_convert_type(i, jnp.float32)
    xh = 0.5 * x
    y = y * (1.5 - xh * y * y)
    y = y * (1.5 - xh * y * y)
    y = y * (1.5 - xh * y * y)
    return y


_GDN = lax.GatherDimensionNumbers(
    offset_dims=(), collapsed_slice_dims=(0,), start_index_map=(0,))


def _shuffle(v, p):
    return lax.gather(v, p[:, None], _GDN, (1,),
                      mode=lax.GatherScatterMode.PROMISE_IN_BOUNDS)


def _lane_sum(v, perms):
    for p in perms:
        v = v + _shuffle(v, p)
    return v


def _edge_kernel_body(ge_hbm, s_hbm, idx_hbm, src_hbm, gam_hbm, bet_hbm,
                      out_hbm, idxv, srcv, ge_rows, s_rows, gv, bv, sem):
    cid = lax.axis_index("c")
    sid = lax.axis_index("s")
    wid = sid * 2 + cid
    pltpu.sync_copy(gam_hbm, gv)
    pltpu.sync_copy(bet_hbm, bv)

    base = wid * RPW
    inv_d = 1.0 / D
    lanes = lax.iota(jnp.int32, 16)
    perms = [lax.bitwise_xor(lanes, jnp.int32(k)) for k in (8, 4, 2, 1)]

    def slab(k, _):
        pltpu.sync_copy(idx_hbm.at[pl.ds(base + 8 * k, 8)], idxv)
        pltpu.sync_copy(src_hbm.at[pl.ds(base + 8 * k, 8)], srcv)

        def sub(j, _):
            cp0 = pltpu.async_copy(ge_hbm.at[idxv.at[j]], ge_rows, sem)
            cp1 = pltpu.async_copy(s_hbm.at[srcv.at[j]], s_rows, sem)
            cp0.wait()
            cp1.wait()

            def row(r, _):
                acc = jnp.zeros((16,), jnp.float32)
                for q in range(8):
                    sl = pl.ds(16 * q, 16)
                    v = ge_rows[r, sl] + s_rows[r, sl]
                    s_rows[r, sl] = v
                    acc = acc + v
                mu = _lane_sum(acc, perms) * inv_d
                acc2 = jnp.zeros((16,), jnp.float32)
                for q in range(8):
                    d = s_rows[r, pl.ds(16 * q, 16)] - mu
                    acc2 = acc2 + d * d
                rstd = _newton_rsqrt(_lane_sum(acc2, perms) * inv_d + 1e-5)
                for q in range(8):
                    sl = pl.ds(16 * q, 16)
                    o = (s_rows[r, sl] - mu) * rstd * gv[0, sl] + bv[0, sl]
                    ge_rows[r, sl] = jnp.maximum(o, 0.0)
                return 0

            lax.fori_loop(0, 128, row, 0)
            pltpu.sync_copy(
                ge_rows,
                out_hbm.at[pl.ds(wid * EPW + k * 1024 + j * 128, 128)])
            return 0

        lax.fori_loop(0, 8, sub, 0)
        return 0

    lax.fori_loop(0, NSLAB, slab, 0)


_gather_call = functools.partial(
    pl.kernel,
    out_type=[jax.ShapeDtypeStruct((NSEG, D), jnp.float32),
              jax.ShapeDtypeStruct((NSEG, D), jnp.float32)],
    mesh=_SC_MESH,
    scratch_types=[pltpu.VMEM((8, 128), jnp.int32),
                   pltpu.VMEM((128, D), jnp.float32),
                   pltpu.SemaphoreType.DMA],
)(_gather_kernel_body)

_scatter_call = functools.partial(
    pl.kernel,
    out_type=jax.ShapeDtypeStruct((2, NSEG, D), jnp.float32),
    mesh=_SC_MESH,
    scratch_types=[pltpu.VMEM_SHARED((NSEG, D), jnp.float32),
                   pltpu.VMEM((8, 128), jnp.int32),
                   pltpu.VMEM((8, 128), jnp.int32),
                   pltpu.VMEM((128, D), jnp.float32),
                   pltpu.SemaphoreType.DMA],
)(_scatter_kernel_body)

_edge_call = functools.partial(
    pl.kernel,
    out_type=jax.ShapeDtypeStruct((EPAD, D), jnp.float32),
    mesh=_SC_MESH,
    scratch_types=[pltpu.VMEM((8, 128), jnp.int32),
                   pltpu.VMEM((8, 128), jnp.int32),
                   pltpu.VMEM((128, D), jnp.float32),
                   pltpu.VMEM((128, D), jnp.float32),
                   pltpu.VMEM((1, D), jnp.float32),
                   pltpu.VMEM((1, D), jnp.float32),
                   pltpu.SemaphoreType.DMA],
)(_edge_kernel_body)


def kernel(graph_x, graph_edge_attr, buffer_edge_index, buffer_edge_attr,
           buffer_batch, comp, idx, mask, W, b, ln_gamma, ln_beta):
    w3 = W.T * (1.0 / 3.0)

    gex = _matmul(jnp.concatenate([graph_edge_attr, graph_x], axis=0), w3)
    ge = gex[:ECO]
    xw = gex[ECO:]

    npad_n = NSEG - NFI
    comp_p = jnp.concatenate(
        [comp, jnp.arange(npad_n, dtype=jnp.int32) % NCO]).reshape(80, 128)

    npad_e = EPAD - EFI
    fill = jnp.arange(npad_e, dtype=jnp.int32)
    idx_p = jnp.concatenate([idx, fill % ECO]).reshape(EROWS, 128)
    src_p = jnp.concatenate(
        [buffer_edge_index[0], fill % NSEG]).reshape(EROWS, 128)
    dst_p = jnp.concatenate(
        [buffer_edge_index[1], NFI + fill % (NSEG - NFI)]).reshape(EROWS, 128)

    x_pad, xc = _gather_call(graph_x, xw, comp_p)

    zseg = jnp.zeros((NSEG, D), jnp.float32)
    segcnt = _scatter_call(ge, idx_p, dst_p, zseg)

    s_tab = _combine(segcnt, xc, b.reshape(1, D))

    out_pad = _edge_call(ge, s_tab, idx_p, src_p,
                         ln_gamma.reshape(1, D), ln_beta.reshape(1, D))

    return (x_pad[:NFI], out_pad[:EFI], buffer_edge_index, buffer_batch)


# speedup vs baseline: 4.5712x; 1.6544x over previous
import functools

import jax
import jax.numpy as jnp
from jax import lax
from jax.experimental import pallas as pl
from jax.experimental.pallas import tpu as pltpu
from jax.experimental.pallas import tpu_sc as plsc

NCO, NFI, EFI, ECO, D = 5000, 10000, 320000, 160000, 128
NSEG = 10240
EPAD = 327680
NW = 32
EPW = EPAD // NW
EROWS = EPAD // 128
RPW = EPW // 128
NSLAB = RPW // 8
NRT = NSEG // 16


def _mm_body(a_ref, w_ref, o_ref):
    o_ref[...] = jnp.dot(a_ref[...], w_ref[...],
                         preferred_element_type=jnp.float32,
                         precision=lax.Precision.HIGHEST)


def _matmul(a, w):
    m = a.shape[0]
    blk = 1000
    return pl.pallas_call(
        _mm_body,
        grid=(m // blk,),
        in_specs=[pl.BlockSpec((blk, D), lambda i: (i, 0)),
                  pl.BlockSpec((D, D), lambda i: (0, 0))],
        out_specs=pl.BlockSpec((blk, D), lambda i: (i, 0)),
        out_shape=jax.ShapeDtypeStruct((m, D), jnp.float32),
    )(a, w)


def _combine_body(sc_ref, xc_ref, b_ref, o_ref):
    cnt = jnp.maximum(sc_ref[1, :, 0:1], 1.0)
    o_ref[...] = sc_ref[0] / cnt + xc_ref[...] + b_ref[...]


def _combine(segcnt, xc, b2):
    blk = 1024
    return pl.pallas_call(
        _combine_body,
        grid=(NSEG // blk,),
        in_specs=[pl.BlockSpec((2, blk, D), lambda i: (0, i, 0)),
                  pl.BlockSpec((blk, D), lambda i: (i, 0)),
                  pl.BlockSpec((1, D), lambda i: (0, 0))],
        out_specs=pl.BlockSpec((blk, D), lambda i: (i, 0)),
        out_shape=jax.ShapeDtypeStruct((NSEG, D), jnp.float32),
    )(segcnt, xc, b2)


_SC_MESH = plsc.VectorSubcoreMesh(core_axis_name="c", subcore_axis_name="s")


def _gather_kernel_body(gx_hbm, xw_hbm, comp_hbm, xout_hbm, xc_hbm,
                        idxv, rows, sem):
    cid = lax.axis_index("c")
    sid = lax.axis_index("s")
    wid = sid * 2 + cid
    t = wid // 16
    u = wid % 16

    def gather_to(tab, out):
        pltpu.sync_copy(comp_hbm.at[pl.ds(8 * u, 8)], idxv)

        def body(j, _):
            pltpu.async_copy(tab.at[idxv.at[j]], rows, sem).wait()
            pltpu.sync_copy(rows, out.at[pl.ds(u * 1024 + j * 128, 128)])
            return 0
        lax.fori_loop(0, 8, body, 0)

    @pl.when(jnp.logical_and(t == 0, u < 10))
    def _():
        gather_to(gx_hbm, xout_hbm)

    @pl.when(jnp.logical_and(t == 1, u < 10))
    def _():
        gather_to(xw_hbm, xc_hbm)


def _scatter_kernel_body(ge_hbm, idx_hbm, dst_hbm, zseg_hbm, out_hbm,
                         tab_sh, idxv, dstv, rows, sem):
    cid = lax.axis_index("c")
    sid = lax.axis_index("s")
    r0 = sid * NRT
    pltpu.sync_copy(zseg_hbm.at[pl.ds(r0, NRT)], tab_sh.at[pl.ds(r0, NRT)])

    @pl.when(cid == 1)
    def _():
        one = jnp.ones((16,), jnp.float32)

        def orow(i, _):
            rows[lax.shift_right_logical(i, 3),
                 pl.ds(lax.mul(lax.rem(i, 8), 16), 16)] = one
            return 0
        lax.fori_loop(0, 1024, orow, 0)

    plsc.subcore_barrier()

    base = sid * (EROWS // 16)

    @pl.when(cid == 0)
    def _():
        def slab(k, _):
            pltpu.sync_copy(idx_hbm.at[pl.ds(base + 16 * k, 16)], idxv)
            pltpu.sync_copy(dst_hbm.at[pl.ds(base + 16 * k, 16)], dstv)
            pend = pltpu.async_copy(ge_hbm.at[idxv.at[0]],
                                    rows.at[pl.ds(0, 128)], sem)
            for j in range(16):
                h = (j & 1) * 128
                pend.wait()
                if j < 15:
                    pend = pltpu.async_copy(
                        ge_hbm.at[idxv.at[j + 1]],
                        rows.at[pl.ds(128 - h, 128)], sem)
                pltpu.sync_copy(rows.at[pl.ds(h, 128)],
                                tab_sh.at[dstv.at[j]], add=True)
            return 0

        lax.fori_loop(0, EROWS // 256, slab, 0)

    @pl.when(cid == 1)
    def _():
        def slab(k, _):
            pltpu.sync_copy(dst_hbm.at[pl.ds(base + 16 * k, 16)], dstv)

            def sub(j, _):
                pltpu.sync_copy(rows.at[pl.ds(0, 128)],
                                tab_sh.at[dstv.at[j]], add=True)
                return 0

            lax.fori_loop(0, 16, sub, 0)
            return 0

        lax.fori_loop(0, EROWS // 256, slab, 0)

    plsc.subcore_barrier()
    pltpu.sync_copy(tab_sh.at[pl.ds(r0, NRT)], out_hbm.at[cid, pl.ds(r0, NRT)])


def _newton_rsqrt(x):
    i = lax.bitcast_convert_type(x, jnp.int32)
    i = jnp.int32(0x5F3759DF) - lax.shift_right_logical(i, 1)
    y = lax.bitcast_convert_type(i, jnp.float32)
    xh = 0.5 * x
    y = y * (1.5 - xh * y * y)
    y = y * (1.5 - xh * y * y)
    y = y * (1.5 - xh * y * y)
    return y


_GDN = lax.GatherDimensionNumbers(
    offset_dims=(), collapsed_slice_dims=(0,), start_index_map=(0,))


def _shuffle(v, p):
    return lax.gather(v, p[:, None], _GDN, (1,),
                      mode=lax.GatherScatterMode.PROMISE_IN_BOUNDS)


def _lane_sum(v, perms):
    for p in perms:
        v = v + _shuffle(v, p)
    return v


def _edge_kernel_body(ge_hbm, s_hbm, idx_hbm, src_hbm, gam_hbm, bet_hbm,
                      out_hbm, idxv, srcv, ge_rows, s_rows, gv, bv, sem, sem2):
    cid = lax.axis_index("c")
    sid = lax.axis_index("s")
    wid = sid * 2 + cid
    pltpu.sync_copy(gam_hbm, gv)
    pltpu.sync_copy(bet_hbm, bv)

    base = wid * RPW
    inv_d = 1.0 / D
    lanes = lax.iota(jnp.int32, 16)
    perms = [lax.bitwise_xor(lanes, jnp.int32(k)) for k in (8, 4, 2, 1)]
    g_regs = [gv[0, pl.ds(16 * q, 16)] for q in range(8)]
    b_regs = [bv[0, pl.ds(16 * q, 16)] for q in range(8)]

    def fire(j, idxv_, srcv_):
        h = (j & 1) * 128
        cg = pltpu.async_copy(ge_hbm.at[idxv_.at[j]],
                              ge_rows.at[pl.ds(h, 128)], sem)
        cs = pltpu.async_copy(s_hbm.at[srcv_.at[j]],
                              s_rows.at[pl.ds(h, 128)], sem)
        return cg, cs

    def slab(k, _):
        pltpu.sync_copy(idx_hbm.at[pl.ds(base + 16 * k, 16)], idxv)
        pltpu.sync_copy(src_hbm.at[pl.ds(base + 16 * k, 16)], srcv)
        pend = fire(0, idxv, srcv)
        outs = [None] * 16
        for j in range(16):
            h = (j & 1) * 128
            pend[0].wait()
            pend[1].wait()
            if j < 15:
                if j >= 1:
                    outs[j - 1].wait()
                pend = fire(j + 1, idxv, srcv)

            def row(r, _):
                v = [ge_rows[h + r, pl.ds(16 * q, 16)]
                     + s_rows[h + r, pl.ds(16 * q, 16)] for q in range(8)]
                acc = ((v[0] + v[1]) + (v[2] + v[3])) \
                    + ((v[4] + v[5]) + (v[6] + v[7]))
                w = [t * t for t in v]
                acc2 = ((w[0] + w[1]) + (w[2] + w[3])) \
                    + ((w[4] + w[5]) + (w[6] + w[7]))
                mu = _lane_sum(acc, perms) * inv_d
                m2 = _lane_sum(acc2, perms) * inv_d
                rstd = _newton_rsqrt(m2 - mu * mu + 1e-5)
                for q in range(8):
                    o = (v[q] - mu) * rstd * g_regs[q] + b_regs[q]
                    ge_rows[h + r, pl.ds(16 * q, 16)] = jnp.maximum(o, 0.0)
                return 0

            lax.fori_loop(0, 128, row, 0)
            outs[j] = pltpu.async_copy(
                ge_rows.at[pl.ds(h, 128)],
                out_hbm.at[pl.ds(wid * EPW + k * 2048 + j * 128, 128)],
                sem2)
        outs[14].wait()
        outs[15].wait()
        return 0

    lax.fori_loop(0, RPW // 16, slab, 0)


_gather_call = functools.partial(
    pl.kernel,
    out_type=[jax.ShapeDtypeStruct((NSEG, D), jnp.float32),
              jax.ShapeDtypeStruct((NSEG, D), jnp.float32)],
    mesh=_SC_MESH,
    scratch_types=[pltpu.VMEM((8, 128), jnp.int32),
                   pltpu.VMEM((128, D), jnp.float32),
                   pltpu.SemaphoreType.DMA],
)(_gather_kernel_body)

_scatter_call = functools.partial(
    pl.kernel,
    out_type=jax.ShapeDtypeStruct((2, NSEG, D), jnp.float32),
    mesh=_SC_MESH,
    scratch_types=[pltpu.VMEM_SHARED((NSEG, D), jnp.float32),
                   pltpu.VMEM((16, 128), jnp.int32),
                   pltpu.VMEM((16, 128), jnp.int32),
                   pltpu.VMEM((256, D), jnp.float32),
                   pltpu.SemaphoreType.DMA],
)(_scatter_kernel_body)

_edge_call = functools.partial(
    pl.kernel,
    out_type=jax.ShapeDtypeStruct((EPAD, D), jnp.float32),
    mesh=_SC_MESH,
    scratch_types=[pltpu.VMEM((16, 128), jnp.int32),
                   pltpu.VMEM((16, 128), jnp.int32),
                   pltpu.VMEM((256, D), jnp.float32),
                   pltpu.VMEM((256, D), jnp.float32),
                   pltpu.VMEM((1, D), jnp.float32),
                   pltpu.VMEM((1, D), jnp.float32),
                   pltpu.SemaphoreType.DMA,
                   pltpu.SemaphoreType.DMA],
)(_edge_kernel_body)


def kernel(graph_x, graph_edge_attr, buffer_edge_index, buffer_edge_attr,
           buffer_batch, comp, idx, mask, W, b, ln_gamma, ln_beta):
    w3 = W.T * (1.0 / 3.0)

    gex = _matmul(jnp.concatenate([graph_edge_attr, graph_x], axis=0), w3)
    ge = gex[:ECO]
    xw = gex[ECO:]

    npad_n = NSEG - NFI
    comp_p = jnp.concatenate(
        [comp, jnp.arange(npad_n, dtype=jnp.int32) % NCO]).reshape(80, 128)

    npad_e = EPAD - EFI
    fill = jnp.arange(npad_e, dtype=jnp.int32)
    idx_p = jnp.concatenate([idx, fill % ECO]).reshape(EROWS, 128)
    src_p = jnp.concatenate(
        [buffer_edge_index[0], fill % NSEG]).reshape(EROWS, 128)
    dst_p = jnp.concatenate(
        [buffer_edge_index[1], NFI + fill % (NSEG - NFI)]).reshape(EROWS, 128)

    x_pad, xc = _gather_call(graph_x, xw, comp_p)

    zseg = jnp.zeros((NSEG, D), jnp.float32)
    segcnt = _scatter_call(ge, idx_p, dst_p, zseg)

    s_tab = _combine(segcnt, xc, b.reshape(1, D))

    out_pad = _edge_call(ge, s_tab, idx_p, src_p,
                         ln_gamma.reshape(1, D), ln_beta.reshape(1, D))

    return (x_pad[:NFI], out_pad[:EFI], buffer_edge_index, buffer_batch)


# speedup vs baseline: 5.4108x; 1.1837x over previous
import functools

import jax
import jax.numpy as jnp
from jax import lax
from jax.experimental import pallas as pl
from jax.experimental.pallas import tpu as pltpu
from jax.experimental.pallas import tpu_sc as plsc

NCO, NFI, EFI, ECO, D = 5000, 10000, 320000, 160000, 128
NSEG = 10240
EPAD = 327680
NW = 32
EPW = EPAD // NW
EROWS = EPAD // 128
RPW = EPW // 128
NSLAB = RPW // 8
NRT = NSEG // 16


def _mm_body(a_ref, w_ref, o_ref):
    o_ref[...] = jnp.dot(a_ref[...], w_ref[...],
                         preferred_element_type=jnp.float32,
                         precision=lax.Precision.HIGHEST)


def _matmul(a, w):
    m = a.shape[0]
    blk = 1000
    return pl.pallas_call(
        _mm_body,
        grid=(m // blk,),
        in_specs=[pl.BlockSpec((blk, D), lambda i: (i, 0)),
                  pl.BlockSpec((D, D), lambda i: (0, 0))],
        out_specs=pl.BlockSpec((blk, D), lambda i: (i, 0)),
        out_shape=jax.ShapeDtypeStruct((m, D), jnp.float32),
    )(a, w)


def _combine_body(sc_ref, xc_ref, b_ref, o_ref):
    cnt = jnp.maximum(sc_ref[1, :, 0:1], 1.0)
    o_ref[...] = sc_ref[0] / cnt + xc_ref[...] + b_ref[...]


def _combine(segcnt, xc, b2):
    blk = 1024
    return pl.pallas_call(
        _combine_body,
        grid=(NSEG // blk,),
        in_specs=[pl.BlockSpec((2, blk, D), lambda i: (0, i, 0)),
                  pl.BlockSpec((blk, D), lambda i: (i, 0)),
                  pl.BlockSpec((1, D), lambda i: (0, 0))],
        out_specs=pl.BlockSpec((blk, D), lambda i: (i, 0)),
        out_shape=jax.ShapeDtypeStruct((NSEG, D), jnp.float32),
    )(segcnt, xc, b2)


_SC_MESH = plsc.VectorSubcoreMesh(core_axis_name="c", subcore_axis_name="s")


def _gather_kernel_body(gx_hbm, xw_hbm, comp_hbm, xout_hbm, xc_hbm,
                        idxv, rows, sem):
    cid = lax.axis_index("c")
    sid = lax.axis_index("s")
    wid = sid * 2 + cid
    t = wid // 16
    u = wid % 16

    def gather_to(tab, out):
        pltpu.sync_copy(comp_hbm.at[pl.ds(8 * u, 8)], idxv)

        def body(j, _):
            pltpu.async_copy(tab.at[idxv.at[j]], rows, sem).wait()
            pltpu.sync_copy(rows, out.at[pl.ds(u * 1024 + j * 128, 128)])
            return 0
        lax.fori_loop(0, 8, body, 0)

    @pl.when(jnp.logical_and(t == 0, u < 10))
    def _():
        gather_to(gx_hbm, xout_hbm)

    @pl.when(jnp.logical_and(t == 1, u < 10))
    def _():
        gather_to(xw_hbm, xc_hbm)


def _scatter_kernel_body(ge_hbm, idx_hbm, dst_hbm, zseg_hbm, out_hbm,
                         tab_sh, idxv, dstv, rows, sem):
    cid = lax.axis_index("c")
    sid = lax.axis_index("s")
    r0 = sid * NRT
    pltpu.sync_copy(zseg_hbm.at[pl.ds(r0, NRT)], tab_sh.at[pl.ds(r0, NRT)])

    @pl.when(cid == 1)
    def _():
        one = jnp.ones((16,), jnp.float32)

        def orow(i, _):
            rows[lax.shift_right_logical(i, 3),
                 pl.ds(lax.mul(lax.rem(i, 8), 16), 16)] = one
            return 0
        lax.fori_loop(0, 1024, orow, 0)

    plsc.subcore_barrier()

    base = sid * (EROWS // 16)

    @pl.when(cid == 0)
    def _():
        def slab(k, _):
            pltpu.sync_copy(idx_hbm.at[pl.ds(base + 16 * k, 16)], idxv)
            pltpu.sync_copy(dst_hbm.at[pl.ds(base + 16 * k, 16)], dstv)
            pend = pltpu.async_copy(ge_hbm.at[idxv.at[0]],
                                    rows.at[pl.ds(0, 128)], sem)
            for j in range(16):
                h = (j & 1) * 128
                pend.wait()
                if j < 15:
                    pend = pltpu.async_copy(
                        ge_hbm.at[idxv.at[j + 1]],
                        rows.at[pl.ds(128 - h, 128)], sem)
                pltpu.sync_copy(rows.at[pl.ds(h, 128)],
                                tab_sh.at[dstv.at[j]], add=True)
            return 0

        lax.fori_loop(0, EROWS // 256, slab, 0)

    @pl.when(cid == 1)
    def _():
        def slab(k, _):
            pltpu.sync_copy(dst_hbm.at[pl.ds(base + 16 * k, 16)], dstv)

            def sub(j, _):
                pltpu.sync_copy(rows.at[pl.ds(0, 128)],
                                tab_sh.at[dstv.at[j]], add=True)
                return 0

            lax.fori_loop(0, 16, sub, 0)
            return 0

        lax.fori_loop(0, EROWS // 256, slab, 0)

    plsc.subcore_barrier()
    pltpu.sync_copy(tab_sh.at[pl.ds(r0, NRT)], out_hbm.at[cid, pl.ds(r0, NRT)])


def _newton_rsqrt(x):
    i = lax.bitcast_convert_type(x, jnp.int32)
    i = jnp.int32(0x5F3759DF) - lax.shift_right_logical(i, 1)
    y = lax.bitcast_convert_type(i, jnp.float32)
    xh = 0.5 * x
    y = y * (1.5 - xh * y * y)
    y = y * (1.5 - xh * y * y)
    y = y * (1.5 - xh * y * y)
    return y


_GDN = lax.GatherDimensionNumbers(
    offset_dims=(), collapsed_slice_dims=(0,), start_index_map=(0,))


def _shuffle(v, p):
    return lax.gather(v, p[:, None], _GDN, (1,),
                      mode=lax.GatherScatterMode.PROMISE_IN_BOUNDS)


def _lane_sum(v, perms):
    for p in perms:
        v = v + _shuffle(v, p)
    return v


def _edge_kernel_body(ge_hbm, s_hbm, idx_hbm, src_hbm, gam_hbm, bet_hbm,
                      out_hbm, idxv, srcv, ge_rows, s_rows, gv, bv, sem, sem2):
    cid = lax.axis_index("c")
    sid = lax.axis_index("s")
    wid = sid * 2 + cid
    pltpu.sync_copy(gam_hbm, gv)
    pltpu.sync_copy(bet_hbm, bv)

    base = wid * RPW
    inv_d = 1.0 / D
    lanes = lax.iota(jnp.int32, 16)
    perms = [lax.bitwise_xor(lanes, jnp.int32(k)) for k in (8, 4, 2, 1)]
    g_regs = [gv[0, pl.ds(16 * q, 16)] for q in range(8)]
    b_regs = [bv[0, pl.ds(16 * q, 16)] for q in range(8)]

    def fire(j, idxv_, srcv_):
        h = (j & 1) * 128
        cg = pltpu.async_copy(ge_hbm.at[idxv_.at[j]],
                              ge_rows.at[pl.ds(h, 128)], sem)
        cs = pltpu.async_copy(s_hbm.at[srcv_.at[j]],
                              s_rows.at[pl.ds(h, 128)], sem)
        return cg, cs

    def slab(k, _):
        pltpu.sync_copy(idx_hbm.at[pl.ds(base + 16 * k, 16)], idxv)
        pltpu.sync_copy(src_hbm.at[pl.ds(base + 16 * k, 16)], srcv)
        pend = fire(0, idxv, srcv)
        outs = [None] * 16
        for j in range(16):
            h = (j & 1) * 128
            pend[0].wait()
            pend[1].wait()
            if j < 15:
                if j >= 1:
                    outs[j - 1].wait()
                pend = fire(j + 1, idxv, srcv)

            def row(r, _):
                v = [ge_rows[h + r, pl.ds(16 * q, 16)]
                     + s_rows[h + r, pl.ds(16 * q, 16)] for q in range(8)]
                acc = ((v[0] + v[1]) + (v[2] + v[3])) \
                    + ((v[4] + v[5]) + (v[6] + v[7]))
                w = [t * t for t in v]
                acc2 = ((w[0] + w[1]) + (w[2] + w[3])) \
                    + ((w[4] + w[5]) + (w[6] + w[7]))
                mu = _lane_sum(acc, perms) * inv_d
                m2 = _lane_sum(acc2, perms) * inv_d
                rstd = _newton_rsqrt(m2 - mu * mu + 1e-5)
                for q in range(8):
                    o = (v[q] - mu) * rstd * g_regs[q] + b_regs[q]
                    ge_rows[h + r, pl.ds(16 * q, 16)] = jnp.maximum(o, 0.0)
                return 0

            lax.fori_loop(0, 128, row, 0)
            off = jnp.minimum(wid * EPW + k * 2048 + j * 128, EFI - 128)
            outs[j] = pltpu.async_copy(
                ge_rows.at[pl.ds(h, 128)], out_hbm.at[pl.ds(off, 128)], sem2)
        outs[14].wait()
        outs[15].wait()
        return 0

    lax.fori_loop(0, RPW // 16, slab, 0)


_gather_call = functools.partial(
    pl.kernel,
    out_type=[jax.ShapeDtypeStruct((NSEG, D), jnp.float32),
              jax.ShapeDtypeStruct((NSEG, D), jnp.float32)],
    mesh=_SC_MESH,
    scratch_types=[pltpu.VMEM((8, 128), jnp.int32),
                   pltpu.VMEM((128, D), jnp.float32),
                   pltpu.SemaphoreType.DMA],
)(_gather_kernel_body)

_scatter_call = functools.partial(
    pl.kernel,
    out_type=jax.ShapeDtypeStruct((2, NSEG, D), jnp.float32),
    mesh=_SC_MESH,
    scratch_types=[pltpu.VMEM_SHARED((NSEG, D), jnp.float32),
                   pltpu.VMEM((16, 128), jnp.int32),
                   pltpu.VMEM((16, 128), jnp.int32),
                   pltpu.VMEM((256, D), jnp.float32),
                   pltpu.SemaphoreType.DMA],
)(_scatter_kernel_body)

_edge_call = functools.partial(
    pl.kernel,
    out_type=jax.ShapeDtypeStruct((EFI, D), jnp.float32),
    mesh=_SC_MESH,
    scratch_types=[pltpu.VMEM((16, 128), jnp.int32),
                   pltpu.VMEM((16, 128), jnp.int32),
                   pltpu.VMEM((256, D), jnp.float32),
                   pltpu.VMEM((256, D), jnp.float32),
                   pltpu.VMEM((1, D), jnp.float32),
                   pltpu.VMEM((1, D), jnp.float32),
                   pltpu.SemaphoreType.DMA,
                   pltpu.SemaphoreType.DMA],
)(_edge_kernel_body)


def kernel(graph_x, graph_edge_attr, buffer_edge_index, buffer_edge_attr,
           buffer_batch, comp, idx, mask, W, b, ln_gamma, ln_beta):
    w3 = W.T * (1.0 / 3.0)

    ge = _matmul(graph_edge_attr, w3)
    xw = _matmul(graph_x, w3)

    npad_n = NSEG - NFI
    comp_p = jnp.concatenate(
        [comp, jnp.arange(npad_n, dtype=jnp.int32) % NCO]).reshape(80, 128)

    npad_e = EPAD - EFI
    fill = jnp.arange(npad_e, dtype=jnp.int32)
    tail = EFI - 128 + (fill % 128)
    idx_p = jnp.concatenate([idx, idx[tail]]).reshape(EROWS, 128)
    src_p = jnp.concatenate(
        [buffer_edge_index[0], buffer_edge_index[0][tail]]).reshape(EROWS, 128)
    dst_p = jnp.concatenate(
        [buffer_edge_index[1], NFI + fill % (NSEG - NFI)]).reshape(EROWS, 128)

    x_pad, xc = _gather_call(graph_x, xw, comp_p)

    zseg = jnp.zeros((NSEG, D), jnp.float32)
    segcnt = _scatter_call(ge, idx_p, dst_p, zseg)

    s_tab = _combine(segcnt, xc, b.reshape(1, D))

    out_pad = _edge_call(ge, s_tab, idx_p, src_p,
                         ln_gamma.reshape(1, D), ln_beta.reshape(1, D))

    return (x_pad[:NFI], out_pad, buffer_edge_index, buffer_batch)


# speedup vs baseline: 6.2908x; 1.1626x over previous
import functools

import jax
import jax.numpy as jnp
from jax import lax
from jax.experimental import pallas as pl
from jax.experimental.pallas import tpu as pltpu
from jax.experimental.pallas import tpu_sc as plsc

NCO, NFI, EFI, ECO, D = 5000, 10000, 320000, 160000, 128
NSEG = 10240
EPAD = 327680
NW = 32
EPW = EPAD // NW
EROWS = EPAD // 128
RPW = EPW // 128
NSLAB = RPW // 8
NRT = NSEG // 16


def _mm_body(a_ref, w_ref, o_ref):
    o_ref[...] = jnp.dot(a_ref[...], w_ref[...],
                         preferred_element_type=jnp.float32,
                         precision=lax.Precision.HIGHEST)


def _matmul(a, w):
    m = a.shape[0]
    blk = 1000
    return pl.pallas_call(
        _mm_body,
        grid=(m // blk,),
        in_specs=[pl.BlockSpec((blk, D), lambda i: (i, 0)),
                  pl.BlockSpec((D, D), lambda i: (0, 0))],
        out_specs=pl.BlockSpec((blk, D), lambda i: (i, 0)),
        out_shape=jax.ShapeDtypeStruct((m, D), jnp.float32),
    )(a, w)


def _combine_body(sc_ref, xc_ref, b_ref, o_ref):
    cnt = jnp.maximum(sc_ref[1, :, 0:1], 1.0)
    o_ref[...] = sc_ref[0] / cnt + xc_ref[...] + b_ref[...]


def _combine(segcnt, xc, b2):
    blk = 1024
    return pl.pallas_call(
        _combine_body,
        grid=(NSEG // blk,),
        in_specs=[pl.BlockSpec((2, blk, D), lambda i: (0, i, 0)),
                  pl.BlockSpec((blk, D), lambda i: (i, 0)),
                  pl.BlockSpec((1, D), lambda i: (0, 0))],
        out_specs=pl.BlockSpec((blk, D), lambda i: (i, 0)),
        out_shape=jax.ShapeDtypeStruct((NSEG, D), jnp.float32),
    )(segcnt, xc, b2)


_SC_MESH = plsc.VectorSubcoreMesh(core_axis_name="c", subcore_axis_name="s")


def _gather_kernel_body(gx_hbm, xw_hbm, comp_hbm, xout_hbm, xc_hbm,
                        idxv, rows, sem):
    cid = lax.axis_index("c")
    sid = lax.axis_index("s")
    wid = sid * 2 + cid
    t = wid // 16
    u = wid % 16

    def gather_to(tab, out):
        pltpu.sync_copy(comp_hbm.at[pl.ds(8 * u, 8)], idxv)

        def body(j, _):
            pltpu.async_copy(tab.at[idxv.at[j]], rows, sem).wait()
            pltpu.sync_copy(rows, out.at[pl.ds(u * 1024 + j * 128, 128)])
            return 0
        lax.fori_loop(0, 8, body, 0)

    @pl.when(jnp.logical_and(t == 0, u < 10))
    def _():
        gather_to(gx_hbm, xout_hbm)

    @pl.when(jnp.logical_and(t == 1, u < 10))
    def _():
        gather_to(xw_hbm, xc_hbm)


def _scatter_kernel_body(ge_hbm, idx_hbm, dst_hbm, zseg_hbm, out_hbm,
                         tab_sh, idxv, dstv, rows, sem):
    cid = lax.axis_index("c")
    sid = lax.axis_index("s")
    r0 = sid * NRT
    pltpu.sync_copy(zseg_hbm.at[pl.ds(r0, NRT)], tab_sh.at[pl.ds(r0, NRT)])

    @pl.when(cid == 1)
    def _():
        one = jnp.ones((16,), jnp.float32)

        def orow(i, _):
            rows[lax.shift_right_logical(i, 3),
                 pl.ds(lax.mul(lax.rem(i, 8), 16), 16)] = one
            return 0
        lax.fori_loop(0, 1024, orow, 0)

    plsc.subcore_barrier()

    base = sid * (EROWS // 16)

    @pl.when(cid == 0)
    def _():
        def slab(k, _):
            pltpu.sync_copy(idx_hbm.at[pl.ds(base + 16 * k, 16)], idxv)
            pltpu.sync_copy(dst_hbm.at[pl.ds(base + 16 * k, 16)], dstv)
            pend = pltpu.async_copy(ge_hbm.at[idxv.at[0]],
                                    rows.at[pl.ds(0, 128)], sem)
            for j in range(16):
                h = (j & 1) * 128
                pend.wait()
                if j < 15:
                    pend = pltpu.async_copy(
                        ge_hbm.at[idxv.at[j + 1]],
                        rows.at[pl.ds(128 - h, 128)], sem)
                pltpu.sync_copy(rows.at[pl.ds(h, 128)],
                                tab_sh.at[dstv.at[j]], add=True)
            return 0

        lax.fori_loop(0, EROWS // 256, slab, 0)

    @pl.when(cid == 1)
    def _():
        def slab(k, _):
            pltpu.sync_copy(dst_hbm.at[pl.ds(base + 16 * k, 16)], dstv)

            def sub(j, _):
                pltpu.sync_copy(rows.at[pl.ds(0, 128)],
                                tab_sh.at[dstv.at[j]], add=True)
                return 0

            lax.fori_loop(0, 16, sub, 0)
            return 0

        lax.fori_loop(0, EROWS // 256, slab, 0)

    plsc.subcore_barrier()
    pltpu.sync_copy(tab_sh.at[pl.ds(r0, NRT)], out_hbm.at[cid, pl.ds(r0, NRT)])


def _newton_rsqrt(x):
    i = lax.bitcast_convert_type(x, jnp.int32)
    i = jnp.int32(0x5F3759DF) - lax.shift_right_logical(i, 1)
    y = lax.bitcast_convert_type(i, jnp.float32)
    xh = 0.5 * x
    y = y * (1.5 - xh * y * y)
    y = y * (1.5 - xh * y * y)
    y = y * (1.5 - xh * y * y)
    return y


_GDN = lax.GatherDimensionNumbers(
    offset_dims=(), collapsed_slice_dims=(0,), start_index_map=(0,))


def _shuffle(v, p):
    return lax.gather(v, p[:, None], _GDN, (1,),
                      mode=lax.GatherScatterMode.PROMISE_IN_BOUNDS)


def _lane_sum(v, perms):
    for p in perms:
        v = v + _shuffle(v, p)
    return v


def _edge_kernel_body(ge_hbm, s_hbm, idx_hbm, src_hbm, gam_hbm, bet_hbm,
                      out_hbm, idxv, srcv, ge_rows, s_rows, gv, bv, sem, sem2):
    cid = lax.axis_index("c")
    sid = lax.axis_index("s")
    wid = sid * 2 + cid
    pltpu.sync_copy(gam_hbm, gv)
    pltpu.sync_copy(bet_hbm, bv)

    base = wid * RPW
    inv_d = 1.0 / D
    lanes = lax.iota(jnp.int32, 16)
    perms = [lax.bitwise_xor(lanes, jnp.int32(k)) for k in (8, 4, 2, 1)]
    g_regs = [gv[0, pl.ds(16 * q, 16)] for q in range(8)]
    b_regs = [bv[0, pl.ds(16 * q, 16)] for q in range(8)]

    def fire(j, idxv_, srcv_):
        h = (j & 1) * 128
        cg = pltpu.async_copy(ge_hbm.at[idxv_.at[j]],
                              ge_rows.at[pl.ds(h, 128)], sem)
        cs = pltpu.async_copy(s_hbm.at[srcv_.at[j]],
                              s_rows.at[pl.ds(h, 128)], sem)
        return cg, cs

    def slab(k, _):
        pltpu.sync_copy(idx_hbm.at[pl.ds(base + 16 * k, 16)], idxv)
        pltpu.sync_copy(src_hbm.at[pl.ds(base + 16 * k, 16)], srcv)
        pend = fire(0, idxv, srcv)
        outs = [None] * 16
        for j in range(16):
            h = (j & 1) * 128
            pend[0].wait()
            pend[1].wait()
            if j < 15:
                if j >= 1:
                    outs[j - 1].wait()
                pend = fire(j + 1, idxv, srcv)

            def one_row(r):
                v = [ge_rows[r, pl.ds(16 * q, 16)]
                     + s_rows[r, pl.ds(16 * q, 16)] for q in range(8)]
                acc = ((v[0] + v[1]) + (v[2] + v[3])) \
                    + ((v[4] + v[5]) + (v[6] + v[7]))
                w = [t * t for t in v]
                acc2 = ((w[0] + w[1]) + (w[2] + w[3])) \
                    + ((w[4] + w[5]) + (w[6] + w[7]))
                mu = _lane_sum(acc, perms) * inv_d
                m2 = _lane_sum(acc2, perms) * inv_d
                rstd = _newton_rsqrt(m2 - mu * mu + 1e-5)
                for q in range(8):
                    o = (v[q] - mu) * rstd * g_regs[q] + b_regs[q]
                    ge_rows[r, pl.ds(16 * q, 16)] = jnp.maximum(o, 0.0)

            def row(r, _):
                one_row(h + 2 * r)
                one_row(h + 2 * r + 1)
                return 0

            lax.fori_loop(0, 64, row, 0)
            off = jnp.minimum(wid * EPW + k * 2048 + j * 128, EFI - 128)
            outs[j] = pltpu.async_copy(
                ge_rows.at[pl.ds(h, 128)], out_hbm.at[pl.ds(off, 128)], sem2)
        outs[14].wait()
        outs[15].wait()
        return 0

    lax.fori_loop(0, RPW // 16, slab, 0)


_gather_call = functools.partial(
    pl.kernel,
    out_type=[jax.ShapeDtypeStruct((NSEG, D), jnp.float32),
              jax.ShapeDtypeStruct((NSEG, D), jnp.float32)],
    mesh=_SC_MESH,
    scratch_types=[pltpu.VMEM((8, 128), jnp.int32),
                   pltpu.VMEM((128, D), jnp.float32),
                   pltpu.SemaphoreType.DMA],
)(_gather_kernel_body)

_scatter_call = functools.partial(
    pl.kernel,
    out_type=jax.ShapeDtypeStruct((2, NSEG, D), jnp.float32),
    mesh=_SC_MESH,
    scratch_types=[pltpu.VMEM_SHARED((NSEG, D), jnp.float32),
                   pltpu.VMEM((16, 128), jnp.int32),
                   pltpu.VMEM((16, 128), jnp.int32),
                   pltpu.VMEM((256, D), jnp.float32),
                   pltpu.SemaphoreType.DMA],
)(_scatter_kernel_body)

_edge_call = functools.partial(
    pl.kernel,
    out_type=jax.ShapeDtypeStruct((EFI, D), jnp.float32),
    mesh=_SC_MESH,
    scratch_types=[pltpu.VMEM((16, 128), jnp.int32),
                   pltpu.VMEM((16, 128), jnp.int32),
                   pltpu.VMEM((256, D), jnp.float32),
                   pltpu.VMEM((256, D), jnp.float32),
                   pltpu.VMEM((1, D), jnp.float32),
                   pltpu.VMEM((1, D), jnp.float32),
                   pltpu.SemaphoreType.DMA,
                   pltpu.SemaphoreType.DMA],
)(_edge_kernel_body)


def kernel(graph_x, graph_edge_attr, buffer_edge_index, buffer_edge_attr,
           buffer_batch, comp, idx, mask, W, b, ln_gamma, ln_beta):
    w3 = W.T * (1.0 / 3.0)

    ge = _matmul(graph_edge_attr, w3)
    xw = _matmul(graph_x, w3)

    npad_n = NSEG - NFI
    comp_p = jnp.concatenate(
        [comp, jnp.arange(npad_n, dtype=jnp.int32) % NCO]).reshape(80, 128)

    npad_e = EPAD - EFI
    fill = jnp.arange(npad_e, dtype=jnp.int32)
    tail = EFI - 128 + (fill % 128)
    idx_p = jnp.concatenate([idx, idx[tail]]).reshape(EROWS, 128)
    src_p = jnp.concatenate(
        [buffer_edge_index[0], buffer_edge_index[0][tail]]).reshape(EROWS, 128)
    dst_p = jnp.concatenate(
        [buffer_edge_index[1], NFI + fill % (NSEG - NFI)]).reshape(EROWS, 128)

    x_pad, xc = _gather_call(graph_x, xw, comp_p)

    zseg = jnp.zeros((NSEG, D), jnp.float32)
    segcnt = _scatter_call(ge, idx_p, dst_p, zseg)

    s_tab = _combine(segcnt, xc, b.reshape(1, D))

    out_pad = _edge_call(ge, s_tab, idx_p, src_p,
                         ln_gamma.reshape(1, D), ln_beta.reshape(1, D))

    return (x_pad[:NFI], out_pad, buffer_edge_index, buffer_batch)
